# baseline (speedup 1.0000x reference)
import functools

import jax
import jax.numpy as jnp
from jax import lax
from jax.experimental import pallas as pl
from jax.experimental.pallas import tpu as pltpu
from jax.experimental.pallas import tpu_sc as plsc

N_NODES = 10000
N_EDGES = 320000
N_QUERY = 100000
D = 128
NEG_SLOPE = 0.2
BN_EPS = 1e-5

NPAD = 10240

NC = 2
NS = 16
NW = NC * NS

EPW = N_EDGES // NW
CH = 80
NCH = EPW // CH

QPAD = 102400
QPW = QPAD // NW
NCHQ = QPW // CH

ROWS_PER_TILE = NPAD // NS


def _proj_body(x_ref, w_ref, avs_ref, avd_ref, xw_ref, as_ref, ad_ref, sv_ref):
    xw = jnp.dot(x_ref[...], w_ref[...], preferred_element_type=jnp.float32)
    xw_ref[...] = xw
    als = jnp.dot(xw, avs_ref[...].reshape(D, 1), preferred_element_type=jnp.float32,
                  precision=lax.Precision.HIGHEST)
    ald = jnp.dot(xw, avd_ref[...].reshape(D, 1), preferred_element_type=jnp.float32,
                  precision=lax.Precision.HIGHEST)
    as_ref[...] = als[:, 0]
    ad_ref[...] = ald[:, 0]
    sv_ref[...] = jnp.full((16,), jnp.max(als) + jnp.max(ald), dtype=jnp.float32)


_proj = pl.pallas_call(
    _proj_body,
    out_shape=[
        jax.ShapeDtypeStruct((NPAD, D), jnp.float32),
        jax.ShapeDtypeStruct((NPAD,), jnp.float32),
        jax.ShapeDtypeStruct((NPAD,), jnp.float32),
        jax.ShapeDtypeStruct((16,), jnp.float32),
    ],
)


def _make_combine(use_bn):
    def body(num_ref, den_ref, b_ref, g_ref, be_ref, o_ref):
        num = num_ref[0] + num_ref[1]
        den = jnp.maximum(den_ref[0] + den_ref[1], 1e-30)
        x = num / den[:, None] + b_ref[...][None, :]
        if use_bn:
            scale = g_ref[...] * (1.0 / jnp.sqrt(1.0 + BN_EPS))
            x = x * scale[None, :] + be_ref[...][None, :]
            x = jnp.maximum(x, 0.0)
        o_ref[...] = x

    return pl.pallas_call(
        body,
        out_shape=jax.ShapeDtypeStruct((NPAD, D), jnp.float32),
    )


_combine_bn = _make_combine(True)
_combine_plain = _make_combine(False)


def _comb_proj_body(num_ref, den_ref, b_ref, g_ref, be_ref,
                    w_ref, avs_ref, avd_ref,
                    xw_ref, as_ref, ad_ref, sv_ref):
    num = num_ref[0] + num_ref[1]
    den = jnp.maximum(den_ref[0] + den_ref[1], 1e-30)
    x = num / den[:, None] + b_ref[...][None, :]
    scale = g_ref[...] * (1.0 / jnp.sqrt(1.0 + BN_EPS))
    x = jnp.maximum(x * scale[None, :] + be_ref[...][None, :], 0.0)
    xw = jnp.dot(x, w_ref[...], preferred_element_type=jnp.float32)
    xw_ref[...] = xw
    als = jnp.dot(xw, avs_ref[...].reshape(D, 1), preferred_element_type=jnp.float32,
                  precision=lax.Precision.HIGHEST)
    ald = jnp.dot(xw, avd_ref[...].reshape(D, 1), preferred_element_type=jnp.float32,
                  precision=lax.Precision.HIGHEST)
    as_ref[...] = als[:, 0]
    ad_ref[...] = ald[:, 0]
    sv_ref[...] = jnp.full((16,), jnp.max(als) + jnp.max(ald), dtype=jnp.float32)


_comb_proj = pl.pallas_call(
    _comb_proj_body,
    out_shape=[
        jax.ShapeDtypeStruct((NPAD, D), jnp.float32),
        jax.ShapeDtypeStruct((NPAD,), jnp.float32),
        jax.ShapeDtypeStruct((NPAD,), jnp.float32),
        jax.ShapeDtypeStruct((16,), jnp.float32),
    ],
)

BQ = 2048


def _mlp_body(h_ref, w1_ref, b1_ref, w2_ref, b2_ref, o_ref):
    h = h_ref[...]
    z = lax.dot_general(h, w1_ref[...], (((1,), (1,)), ((), ())),
                        preferred_element_type=jnp.float32)
    z = jnp.maximum(z + b1_ref[...][None, :], 0.0)
    o = jnp.dot(z, w2_ref[...], preferred_element_type=jnp.float32)
    o = o + b2_ref[0]
    o_ref[...] = 1.0 / (1.0 + jnp.exp(-o))


_mlp = pl.pallas_call(
    _mlp_body,
    grid=(QPAD // BQ,),
    in_specs=[
        pl.BlockSpec((BQ, D), lambda i: (i, 0)),
        pl.BlockSpec((D, D), lambda i: (0, 0)),
        pl.BlockSpec((D,), lambda i: (0,)),
        pl.BlockSpec((D, 1), lambda i: (0, 0)),
        pl.BlockSpec(memory_space=pltpu.SMEM),
    ],
    out_specs=pl.BlockSpec((BQ, 1), lambda i: (i, 0)),
    out_shape=jax.ShapeDtypeStruct((QPAD, 1), jnp.float32),
)


_MESH = plsc.VectorSubcoreMesh(core_axis_name="c", subcore_axis_name="s")


@functools.partial(
    pl.kernel,
    mesh=_MESH,
    compiler_params=pltpu.CompilerParams(needs_layout_passes=False),
    out_type=[
        jax.ShapeDtypeStruct((NC, NPAD, D), jnp.float32),
        jax.ShapeDtypeStruct((NC, NPAD), jnp.float32),
    ],
    scratch_types=[
        pltpu.VMEM((3, CH), jnp.int32),
        pltpu.VMEM((4, CH), jnp.int32),
        pltpu.VMEM((2, CH), jnp.float32),
        pltpu.VMEM((2, CH), jnp.float32),
        pltpu.VMEM((4, CH), jnp.float32),
        pltpu.VMEM((3, CH, D), jnp.float32),
        pltpu.VMEM((16,), jnp.float32),
        pltpu.VMEM((ROWS_PER_TILE,), jnp.float32),
        pltpu.VMEM_SHARED((NPAD, D), jnp.float32),
        pltpu.VMEM_SHARED((NPAD,), jnp.float32),
        pltpu.SemaphoreType.DMA,
        pltpu.SemaphoreType.DMA,
        pltpu.SemaphoreType.DMA,
        pltpu.SemaphoreType.DMA,
        pltpu.SemaphoreType.DMA,
        pltpu.SemaphoreType.DMA,
        pltpu.SemaphoreType.DMA,
    ],
)
def _sc_aggregate(src_hbm, dst_hbm, asrc_hbm, adst_hbm, sv_hbm, xw_hbm,
                  num_out, den_out,
                  src_v, dst_v, avs_v, avd_v, w_v, rows_v, sv_v, zd_v,
                  num_sh, den_sh,
                  sem_si, sem_di, sem_a, sem_d, sem_r, sem_sd, sem_sn):
    cid = lax.axis_index("c")
    sid = lax.axis_index("s")
    wid = sid * NC + cid
    zeros16 = jnp.zeros((16,), jnp.float32)

    def _zb(r, carry):
        for j in range(D // 16):
            rows_v[0, r, pl.ds(j * 16, 16)] = zeros16
        return carry

    lax.fori_loop(0, CH, _zb, 0)
    for g in range(ROWS_PER_TILE // 16):
        zd_v[pl.ds(g * 16, 16)] = zeros16

    row0 = sid * ROWS_PER_TILE

    def zero_num(k):
        return pltpu.make_async_copy(
            rows_v.at[0], num_sh.at[pl.ds(row0 + k * CH, CH), :], sem_sn)

    def zero_den():
        return pltpu.make_async_copy(
            zd_v, den_sh.at[pl.ds(row0, ROWS_PER_TILE)], sem_sd)

    for k in range(ROWS_PER_TILE // CH):
        zero_num(k).start()
    zero_den().start()

    pltpu.sync_copy(sv_hbm, sv_v)
    svv = sv_v[...]

    def idx_src(ci, slot):
        return pltpu.make_async_copy(src_hbm.at[wid, ci], src_v.at[slot], sem_si)

    def idx_dst(ci, slot):
        return pltpu.make_async_copy(dst_hbm.at[wid, ci], dst_v.at[slot], sem_di)

    def g_avs(s3, s2):
        return pltpu.make_async_copy(asrc_hbm.at[src_v.at[s3]], avs_v.at[s2], sem_a)

    def g_avd(s4, s2):
        return pltpu.make_async_copy(adst_hbm.at[dst_v.at[s4]], avd_v.at[s2], sem_d)

    def g_rows(s3):
        return pltpu.make_async_copy(xw_hbm.at[src_v.at[s3]], rows_v.at[s3], sem_r)

    def s_den(s4):
        return pltpu.make_async_copy(w_v.at[s4], den_sh.at[dst_v.at[s4]], sem_sd)

    def s_num(s3, s4):
        return pltpu.make_async_copy(rows_v.at[s3], num_sh.at[dst_v.at[s4]], sem_sn)

    idx_src(0, 0).start()
    idx_dst(0, 0).start()
    idx_src(1, 1).start()
    idx_dst(1, 1).start()
    for k in range(ROWS_PER_TILE // CH):
        zero_num(k).wait()
    zero_den().wait()
    idx_src(0, 0).wait()
    idx_dst(0, 0).wait()
    g_avs(0, 0).start()
    g_avd(0, 0).start()
    g_rows(0).start()
    plsc.subcore_barrier()

    def _chunk(ci, carry):
        s3 = lax.rem(ci, 3)
        s4 = lax.rem(ci, 4)
        s2 = lax.rem(ci, 2)

        @pl.when(ci >= 2)
        def _():
            p3 = lax.rem(ci - 2, 3)
            p4 = lax.rem(ci - 2, 4)
            s_den(p4).wait()
            s_num(p3, p4).wait()

        @pl.when(ci + 2 < NCH)
        def _():
            n3 = lax.rem(ci + 2, 3)
            n4 = lax.rem(ci + 2, 4)
            idx_src(ci + 2, n3).start()
            idx_dst(ci + 2, n4).start()

        @pl.when(ci + 1 < NCH)
        def _():
            n3 = lax.rem(ci + 1, 3)
            n4 = lax.rem(ci + 1, 4)
            n2 = lax.rem(ci + 1, 2)
            idx_src(ci + 1, n3).wait()
            idx_dst(ci + 1, n4).wait()
            g_avs(n3, n2).start()
            g_avd(n4, n2).start()
            g_rows(n3).start()

        g_avs(s3, s2).wait()
        g_avd(s4, s2).wait()
        for g in range(CH // 16):
            e = avs_v[s2, pl.ds(g * 16, 16)] + avd_v[s2, pl.ds(g * 16, 16)]
            e = jnp.where(e >= 0.0, e, e * NEG_SLOPE)
            w_v[s4, pl.ds(g * 16, 16)] = jnp.exp(e - svv)
        s_den(s4).start(add=True)

        g_rows(s3).wait()

        s4v = jnp.full((16,), s4, jnp.int32)

        def _scale(p, c2):
            r0 = p * 2
            w0 = plsc.load_gather(w_v, [s4v, jnp.full((16,), r0, jnp.int32)])
            w1 = plsc.load_gather(w_v, [s4v, jnp.full((16,), r0 + 1, jnp.int32)])
            for j in range(D // 16):
                rows_v[s3, r0, pl.ds(j * 16, 16)] = (
                    rows_v[s3, r0, pl.ds(j * 16, 16)] * w0)
                rows_v[s3, r0 + 1, pl.ds(j * 16, 16)] = (
                    rows_v[s3, r0 + 1, pl.ds(j * 16, 16)] * w1)
            return c2

        lax.fori_loop(0, CH // 2, _scale, 0)
        s_num(s3, s4).start(add=True)
        return carry

    lax.fori_loop(0, NCH, _chunk, 0)

    for ci in (NCH - 2, NCH - 1):
        s_den(ci % 4).wait()
        s_num(ci % 3, ci % 4).wait()

    plsc.subcore_barrier()

    pltpu.sync_copy(num_sh.at[pl.ds(row0, ROWS_PER_TILE), :],
                    num_out.at[cid, pl.ds(row0, ROWS_PER_TILE), :])
    pltpu.sync_copy(den_sh.at[pl.ds(row0, ROWS_PER_TILE)],
                    den_out.at[cid, pl.ds(row0, ROWS_PER_TILE)])


@functools.partial(
    pl.kernel,
    mesh=_MESH,
    compiler_params=pltpu.CompilerParams(needs_layout_passes=False),
    out_type=jax.ShapeDtypeStruct((QPAD, D), jnp.float32),
    scratch_types=[
        pltpu.VMEM((3, CH), jnp.int32),
        pltpu.VMEM((3, CH), jnp.int32),
        pltpu.VMEM((3, CH, D), jnp.float32),
        pltpu.VMEM((3, CH, D), jnp.float32),
        pltpu.SemaphoreType.DMA,
        pltpu.SemaphoreType.DMA,
        pltpu.SemaphoreType.DMA,
        pltpu.SemaphoreType.DMA,
        pltpu.SemaphoreType.DMA,
    ],
)
def _sc_link(e0_hbm, e1_hbm, x_hbm, h_out, e0_v, e1_v, r0_v, r1_v,
             sem_i0, sem_i1, sem_g0, sem_g1, sem_w):
    cid = lax.axis_index("c")
    sid = lax.axis_index("s")
    wid = sid * NC + cid

    def idx0(ci, slot):
        return pltpu.make_async_copy(e0_hbm.at[wid, ci], e0_v.at[slot], sem_i0)

    def idx1(ci, slot):
        return pltpu.make_async_copy(e1_hbm.at[wid, ci], e1_v.at[slot], sem_i1)

    def g0(s3):
        return pltpu.make_async_copy(x_hbm.at[e0_v.at[s3]], r0_v.at[s3], sem_g0)

    def g1(s3):
        return pltpu.make_async_copy(x_hbm.at[e1_v.at[s3]], r1_v.at[s3], sem_g1)

    def wr(ci, s3):
        return pltpu.make_async_copy(
            r0_v.at[s3], h_out.at[pl.ds(wid * QPW + ci * CH, CH), :], sem_w)

    idx0(0, 0).start()
    idx1(0, 0).start()
    idx0(1, 1).start()
    idx1(1, 1).start()
    idx0(0, 0).wait()
    idx1(0, 0).wait()
    g0(0).start()
    g1(0).start()

    def _chunk(ci, carry):
        s3 = lax.rem(ci, 3)

        @pl.when(ci >= 2)
        def _():
            p3 = lax.rem(ci - 2, 3)
            wr(ci - 2, p3).wait()

        @pl.when(ci + 2 < NCHQ)
        def _():
            n3 = lax.rem(ci + 2, 3)
            idx0(ci + 2, n3).start()
            idx1(ci + 2, n3).start()

        @pl.when(ci + 1 < NCHQ)
        def _():
            n3 = lax.rem(ci + 1, 3)
            idx0(ci + 1, n3).wait()
            idx1(ci + 1, n3).wait()
            g0(n3).start()
            g1(n3).start()

        g0(s3).wait()
        g1(s3).wait()

        def _mul(p, c2):
            r0 = p * 2
            for j in range(D // 16):
                r0_v[s3, r0, pl.ds(j * 16, 16)] = (
                    r0_v[s3, r0, pl.ds(j * 16, 16)]
                    * r1_v[s3, r0, pl.ds(j * 16, 16)])
                r0_v[s3, r0 + 1, pl.ds(j * 16, 16)] = (
                    r0_v[s3, r0 + 1, pl.ds(j * 16, 16)]
                    * r1_v[s3, r0 + 1, pl.ds(j * 16, 16)])
            return c2

        lax.fori_loop(0, CH // 2, _mul, 0)
        wr(ci, s3).start()
        return carry

    lax.fori_loop(0, NCHQ, _chunk, 0)

    for ci in (NCHQ - 2, NCHQ - 1):
        wr(ci, ci % 3).wait()


def kernel(edge_index, edges, emb, W1, a_s1, a_d1, b1, g1, be1,
           W2, a_s2, a_d2, b2, pW1, pb1, pW2, pb2):
    src2 = edge_index[0].reshape(NW, NCH, CH)
    dst2 = edge_index[1].reshape(NW, NCH, CH)
    qpad = QPAD - N_QUERY
    e0 = jnp.concatenate([edges[0], jnp.zeros((qpad,), edges.dtype)]).reshape(NW, NCHQ, CH)
    e1 = jnp.concatenate([edges[1], jnp.zeros((qpad,), edges.dtype)]).reshape(NW, NCHQ, CH)
    embp = jnp.pad(emb, ((0, NPAD - N_NODES), (0, 0)))

    xw1, as1, ad1, sv1 = _proj(embp, W1, a_s1, a_d1)
    num1, den1 = _sc_aggregate(src2, dst2, as1, ad1, sv1, xw1)
    xw2, as2, ad2, sv2 = _comb_proj(num1, den1, b1, g1, be1, W2, a_s2, a_d2)
    num2, den2 = _sc_aggregate(src2, dst2, as2, ad2, sv2, xw2)
    x2 = _combine_plain(num2, den2, b2, g1, be1)

    h = _sc_link(e0, e1, x2)
    p = _mlp(h, pW1, pb1, pW2.T, pb2)
    return p[:N_QUERY, 0]


# speedup vs baseline: 1.0687x; 1.0687x over previous
import functools

import jax
import jax.numpy as jnp
from jax import lax
from jax.experimental import pallas as pl
from jax.experimental.pallas import tpu as pltpu
from jax.experimental.pallas import tpu_sc as plsc

N_NODES = 10000
N_EDGES = 320000
N_QUERY = 100000
D = 128
NEG_SLOPE = 0.2
BN_EPS = 1e-5

NPAD = 10240

NC = 2
NS = 16
NW = NC * NS

EPW = N_EDGES // NW
CH = 80
NCH = EPW // CH

QPAD = 102400
QPW = QPAD // NW
NCHQ = QPW // CH

ROWS_PER_TILE = NPAD // NS


def _proj_body(x_ref, w_ref, avs_ref, avd_ref, xw_ref, as_ref, ad_ref, sv_ref):
    xw = jnp.dot(x_ref[...], w_ref[...], preferred_element_type=jnp.float32)
    xw_ref[...] = xw
    als = jnp.dot(xw, avs_ref[...].reshape(D, 1), preferred_element_type=jnp.float32,
                  precision=lax.Precision.HIGHEST)
    ald = jnp.dot(xw, avd_ref[...].reshape(D, 1), preferred_element_type=jnp.float32,
                  precision=lax.Precision.HIGHEST)
    as_ref[...] = als[:, 0]
    ad_ref[...] = ald[:, 0]
    sv_ref[...] = jnp.full((16,), jnp.max(als) + jnp.max(ald), dtype=jnp.float32)


_proj = pl.pallas_call(
    _proj_body,
    out_shape=[
        jax.ShapeDtypeStruct((NPAD, D), jnp.float32),
        jax.ShapeDtypeStruct((NPAD,), jnp.float32),
        jax.ShapeDtypeStruct((NPAD,), jnp.float32),
        jax.ShapeDtypeStruct((16,), jnp.float32),
    ],
)


def _make_combine(use_bn):
    def body(num_ref, den_ref, b_ref, g_ref, be_ref, o_ref):
        num = num_ref[0] + num_ref[1]
        den = jnp.maximum(den_ref[0] + den_ref[1], 1e-30)
        x = num / den[:, None] + b_ref[...][None, :]
        if use_bn:
            scale = g_ref[...] * (1.0 / jnp.sqrt(1.0 + BN_EPS))
            x = x * scale[None, :] + be_ref[...][None, :]
            x = jnp.maximum(x, 0.0)
        o_ref[...] = x

    return pl.pallas_call(
        body,
        out_shape=jax.ShapeDtypeStruct((NPAD, D), jnp.float32),
    )


_combine_bn = _make_combine(True)
_combine_plain = _make_combine(False)

BQ = 2048


def _mlp_body(h_ref, w1_ref, b1_ref, w2_ref, b2_ref, o_ref):
    h = h_ref[...]
    z = lax.dot_general(h, w1_ref[...], (((1,), (1,)), ((), ())),
                        preferred_element_type=jnp.float32)
    z = jnp.maximum(z + b1_ref[...][None, :], 0.0)
    o = jnp.dot(z, w2_ref[...], preferred_element_type=jnp.float32)
    o = o + b2_ref[0]
    o_ref[...] = 1.0 / (1.0 + jnp.exp(-o))


_mlp = pl.pallas_call(
    _mlp_body,
    grid=(QPAD // BQ,),
    in_specs=[
        pl.BlockSpec((BQ, D), lambda i: (i, 0)),
        pl.BlockSpec((D, D), lambda i: (0, 0)),
        pl.BlockSpec((D,), lambda i: (0,)),
        pl.BlockSpec((D, 1), lambda i: (0, 0)),
        pl.BlockSpec(memory_space=pltpu.SMEM),
    ],
    out_specs=pl.BlockSpec((BQ, 1), lambda i: (i, 0)),
    out_shape=jax.ShapeDtypeStruct((QPAD, 1), jnp.float32),
)


_MESH = plsc.VectorSubcoreMesh(core_axis_name="c", subcore_axis_name="s")


@functools.partial(
    pl.kernel,
    mesh=_MESH,
    compiler_params=pltpu.CompilerParams(needs_layout_passes=False),
    out_type=[
        jax.ShapeDtypeStruct((NC, NPAD, D), jnp.float32),
        jax.ShapeDtypeStruct((NC, NPAD), jnp.float32),
    ],
    scratch_types=[
        pltpu.VMEM((3, CH), jnp.int32),
        pltpu.VMEM((4, CH), jnp.int32),
        pltpu.VMEM((2, CH), jnp.float32),
        pltpu.VMEM((2, CH), jnp.float32),
        pltpu.VMEM((4, CH), jnp.float32),
        pltpu.VMEM((3, CH, D), jnp.float32),
        pltpu.VMEM((16,), jnp.float32),
        pltpu.VMEM((ROWS_PER_TILE,), jnp.float32),
        pltpu.VMEM_SHARED((NPAD, D), jnp.float32),
        pltpu.VMEM_SHARED((NPAD,), jnp.float32),
        pltpu.SemaphoreType.DMA,
        pltpu.SemaphoreType.DMA,
        pltpu.SemaphoreType.DMA,
        pltpu.SemaphoreType.DMA,
        pltpu.SemaphoreType.DMA,
        pltpu.SemaphoreType.DMA,
        pltpu.SemaphoreType.DMA,
    ],
)
def _sc_aggregate(src_hbm, dst_hbm, asrc_hbm, adst_hbm, sv_hbm, xw_hbm,
                  num_out, den_out,
                  src_v, dst_v, avs_v, avd_v, w_v, rows_v, sv_v, zd_v,
                  num_sh, den_sh,
                  sem_si, sem_di, sem_a, sem_d, sem_r, sem_sd, sem_sn):
    cid = lax.axis_index("c")
    sid = lax.axis_index("s")
    wid = sid * NC + cid
    zeros16 = jnp.zeros((16,), jnp.float32)

    def _zb(r, carry):
        for j in range(D // 16):
            rows_v[0, r, pl.ds(j * 16, 16)] = zeros16
        return carry

    lax.fori_loop(0, CH, _zb, 0)
    for g in range(ROWS_PER_TILE // 16):
        zd_v[pl.ds(g * 16, 16)] = zeros16

    row0 = sid * ROWS_PER_TILE
    for k in range(ROWS_PER_TILE // CH):
        pltpu.sync_copy(rows_v.at[0], num_sh.at[pl.ds(row0 + k * CH, CH), :])
    pltpu.sync_copy(zd_v, den_sh.at[pl.ds(row0, ROWS_PER_TILE)])

    pltpu.sync_copy(sv_hbm, sv_v)
    svv = sv_v[...]

    plsc.subcore_barrier()

    def idx_src(ci, slot):
        return pltpu.make_async_copy(src_hbm.at[wid, ci], src_v.at[slot], sem_si)

    def idx_dst(ci, slot):
        return pltpu.make_async_copy(dst_hbm.at[wid, ci], dst_v.at[slot], sem_di)

    def g_avs(s3, s2):
        return pltpu.make_async_copy(asrc_hbm.at[src_v.at[s3]], avs_v.at[s2], sem_a)

    def g_avd(s4, s2):
        return pltpu.make_async_copy(adst_hbm.at[dst_v.at[s4]], avd_v.at[s2], sem_d)

    def g_rows(s3):
        return pltpu.make_async_copy(xw_hbm.at[src_v.at[s3]], rows_v.at[s3], sem_r)

    def s_den(s4):
        return pltpu.make_async_copy(w_v.at[s4], den_sh.at[dst_v.at[s4]], sem_sd)

    def s_num(s3, s4):
        return pltpu.make_async_copy(rows_v.at[s3], num_sh.at[dst_v.at[s4]], sem_sn)

    idx_src(0, 0).start()
    idx_dst(0, 0).start()
    idx_src(1, 1).start()
    idx_dst(1, 1).start()
    idx_src(0, 0).wait()
    idx_dst(0, 0).wait()
    g_avs(0, 0).start()
    g_avd(0, 0).start()
    g_rows(0).start()

    def _chunk(ci, carry):
        s3 = lax.rem(ci, 3)
        s4 = lax.rem(ci, 4)
        s2 = lax.rem(ci, 2)

        @pl.when(ci >= 2)
        def _():
            p3 = lax.rem(ci - 2, 3)
            p4 = lax.rem(ci - 2, 4)
            s_den(p4).wait()
            s_num(p3, p4).wait()

        @pl.when(ci + 2 < NCH)
        def _():
            n3 = lax.rem(ci + 2, 3)
            n4 = lax.rem(ci + 2, 4)
            idx_src(ci + 2, n3).start()
            idx_dst(ci + 2, n4).start()

        @pl.when(ci + 1 < NCH)
        def _():
            n3 = lax.rem(ci + 1, 3)
            n4 = lax.rem(ci + 1, 4)
            n2 = lax.rem(ci + 1, 2)
            idx_src(ci + 1, n3).wait()
            idx_dst(ci + 1, n4).wait()
            g_avs(n3, n2).start()
            g_avd(n4, n2).start()
            g_rows(n3).start()

        g_avs(s3, s2).wait()
        g_avd(s4, s2).wait()
        for g in range(CH // 16):
            e = avs_v[s2, pl.ds(g * 16, 16)] + avd_v[s2, pl.ds(g * 16, 16)]
            e = jnp.where(e >= 0.0, e, e * NEG_SLOPE)
            w_v[s4, pl.ds(g * 16, 16)] = jnp.exp(e - svv)
        s_den(s4).start(add=True)

        g_rows(s3).wait()

        s4v = jnp.full((16,), s4, jnp.int32)

        def _scale(p, c2):
            r0 = p * 2
            w0 = plsc.load_gather(w_v, [s4v, jnp.full((16,), r0, jnp.int32)])
            w1 = plsc.load_gather(w_v, [s4v, jnp.full((16,), r0 + 1, jnp.int32)])
            for j in range(D // 16):
                rows_v[s3, r0, pl.ds(j * 16, 16)] = (
                    rows_v[s3, r0, pl.ds(j * 16, 16)] * w0)
                rows_v[s3, r0 + 1, pl.ds(j * 16, 16)] = (
                    rows_v[s3, r0 + 1, pl.ds(j * 16, 16)] * w1)
            return c2

        lax.fori_loop(0, CH // 2, _scale, 0)
        s_num(s3, s4).start(add=True)
        return carry

    lax.fori_loop(0, NCH, _chunk, 0)

    for ci in (NCH - 2, NCH - 1):
        s_den(ci % 4).wait()
        s_num(ci % 3, ci % 4).wait()

    plsc.subcore_barrier()

    pltpu.sync_copy(num_sh.at[pl.ds(row0, ROWS_PER_TILE), :],
                    num_out.at[cid, pl.ds(row0, ROWS_PER_TILE), :])
    pltpu.sync_copy(den_sh.at[pl.ds(row0, ROWS_PER_TILE)],
                    den_out.at[cid, pl.ds(row0, ROWS_PER_TILE)])


@functools.partial(
    pl.kernel,
    mesh=_MESH,
    compiler_params=pltpu.CompilerParams(needs_layout_passes=False),
    out_type=jax.ShapeDtypeStruct((QPAD, D), jnp.float32),
    scratch_types=[
        pltpu.VMEM((3, CH), jnp.int32),
        pltpu.VMEM((3, CH), jnp.int32),
        pltpu.VMEM((3, CH, D), jnp.float32),
        pltpu.VMEM((3, CH, D), jnp.float32),
        pltpu.SemaphoreType.DMA,
        pltpu.SemaphoreType.DMA,
        pltpu.SemaphoreType.DMA,
        pltpu.SemaphoreType.DMA,
        pltpu.SemaphoreType.DMA,
    ],
)
def _sc_link(e0_hbm, e1_hbm, x_hbm, h_out, e0_v, e1_v, r0_v, r1_v,
             sem_i0, sem_i1, sem_g0, sem_g1, sem_w):
    cid = lax.axis_index("c")
    sid = lax.axis_index("s")
    wid = sid * NC + cid

    def idx0(ci, slot):
        return pltpu.make_async_copy(e0_hbm.at[wid, ci], e0_v.at[slot], sem_i0)

    def idx1(ci, slot):
        return pltpu.make_async_copy(e1_hbm.at[wid, ci], e1_v.at[slot], sem_i1)

    def g0(s3):
        return pltpu.make_async_copy(x_hbm.at[e0_v.at[s3]], r0_v.at[s3], sem_g0)

    def g1(s3):
        return pltpu.make_async_copy(x_hbm.at[e1_v.at[s3]], r1_v.at[s3], sem_g1)

    def wr(ci, s3):
        return pltpu.make_async_copy(
            r0_v.at[s3], h_out.at[pl.ds(wid * QPW + ci * CH, CH), :], sem_w)

    idx0(0, 0).start()
    idx1(0, 0).start()
    idx0(1, 1).start()
    idx1(1, 1).start()
    idx0(0, 0).wait()
    idx1(0, 0).wait()
    g0(0).start()
    g1(0).start()

    def _chunk(ci, carry):
        s3 = lax.rem(ci, 3)

        @pl.when(ci >= 2)
        def _():
            p3 = lax.rem(ci - 2, 3)
            wr(ci - 2, p3).wait()

        @pl.when(ci + 2 < NCHQ)
        def _():
            n3 = lax.rem(ci + 2, 3)
            idx0(ci + 2, n3).start()
            idx1(ci + 2, n3).start()

        @pl.when(ci + 1 < NCHQ)
        def _():
            n3 = lax.rem(ci + 1, 3)
            idx0(ci + 1, n3).wait()
            idx1(ci + 1, n3).wait()
            g0(n3).start()
            g1(n3).start()

        g0(s3).wait()
        g1(s3).wait()

        def _mul(p, c2):
            r0 = p * 2
            for j in range(D // 16):
                r0_v[s3, r0, pl.ds(j * 16, 16)] = (
                    r0_v[s3, r0, pl.ds(j * 16, 16)]
                    * r1_v[s3, r0, pl.ds(j * 16, 16)])
                r0_v[s3, r0 + 1, pl.ds(j * 16, 16)] = (
                    r0_v[s3, r0 + 1, pl.ds(j * 16, 16)]
                    * r1_v[s3, r0 + 1, pl.ds(j * 16, 16)])
            return c2

        lax.fori_loop(0, CH // 2, _mul, 0)
        wr(ci, s3).start()
        return carry

    lax.fori_loop(0, NCHQ, _chunk, 0)

    for ci in (NCHQ - 2, NCHQ - 1):
        wr(ci, ci % 3).wait()


def kernel(edge_index, edges, emb, W1, a_s1, a_d1, b1, g1, be1,
           W2, a_s2, a_d2, b2, pW1, pb1, pW2, pb2):
    src2 = edge_index[0].reshape(NW, NCH, CH)
    dst2 = edge_index[1].reshape(NW, NCH, CH)
    qpad = QPAD - N_QUERY
    e0 = jnp.concatenate([edges[0], jnp.zeros((qpad,), edges.dtype)]).reshape(NW, NCHQ, CH)
    e1 = jnp.concatenate([edges[1], jnp.zeros((qpad,), edges.dtype)]).reshape(NW, NCHQ, CH)
    embp = jnp.pad(emb, ((0, NPAD - N_NODES), (0, 0)))

    xw1, as1, ad1, sv1 = _proj(embp, W1, a_s1, a_d1)
    num1, den1 = _sc_aggregate(src2, dst2, as1, ad1, sv1, xw1)
    x1 = _combine_bn(num1, den1, b1, g1, be1)

    xw2, as2, ad2, sv2 = _proj(x1, W2, a_s2, a_d2)
    num2, den2 = _sc_aggregate(src2, dst2, as2, ad2, sv2, xw2)
    x2 = _combine_plain(num2, den2, b2, g1, be1)

    h = _sc_link(e0, e1, x2)
    p = _mlp(h, pW1, pb1, pW2.T, pb2)
    return p[:N_QUERY, 0]
